# Initial kernel scaffold; baseline (speedup 1.0000x reference)
import jax
import jax.numpy as jnp
from jax.experimental import pallas as pl


def kernel(node_attrs, node_feats, edge_attrs, edge_feats, senders, receivers, W_sc, W_lin, W1, W2, W3, W4, W_out):
    raise NotImplementedError("write your pallas kernel here")


# speedup vs baseline: 3.9514x; 3.9514x over previous
import functools

import jax
import jax.numpy as jnp
from jax import lax
from jax.experimental import pallas as pl
from jax.experimental.pallas import tpu as pltpu
from jax.experimental.pallas import tpu_sc as plsc

N_NODES = 10000
N_EDGES = 160000
D = 128
A = 4
HIDDEN = 128
MLP_H = 64
AVG_NUM_NEIGHBORS = 16.0

NC = 2
NS = 16
NW = NC * NS
K = 128
N_CHUNKS = N_EDGES // K
ZROWS = 80
NZCHUNKS = N_NODES // ZROWS

NB = 2000
EB = 4000


def _silu(v):
    return v * jax.nn.sigmoid(v)


def _node_body(nf_ref, na_ref, wsc_ref, wlin_ref, sc_ref, x_ref):
    nf = nf_ref[...]
    na = na_ref[...]
    x_ref[...] = jnp.dot(nf, wlin_ref[...], preferred_element_type=jnp.float32)
    acc = jnp.dot(nf * na[:, 0:1], wsc_ref[0], preferred_element_type=jnp.float32)
    for a in range(1, A):
        acc = acc + jnp.dot(nf * na[:, a:a + 1], wsc_ref[a],
                            preferred_element_type=jnp.float32)
    sc_ref[...] = acc


def _node_call(nf, na, wsc, wlin):
    grid = (N_NODES // NB,)
    return pl.pallas_call(
        _node_body,
        grid=grid,
        in_specs=[
            pl.BlockSpec((NB, D), lambda i: (i, 0)),
            pl.BlockSpec((NB, A), lambda i: (i, 0)),
            pl.BlockSpec((A, D, HIDDEN), lambda i: (0, 0, 0)),
            pl.BlockSpec((D, D), lambda i: (0, 0)),
        ],
        out_specs=[
            pl.BlockSpec((NB, HIDDEN), lambda i: (i, 0)),
            pl.BlockSpec((NB, D), lambda i: (i, 0)),
        ],
        out_shape=[
            jax.ShapeDtypeStruct((N_NODES, HIDDEN), jnp.float32),
            jax.ShapeDtypeStruct((N_NODES, D), jnp.float32),
        ],
    )(nf, na, wsc, wlin)


def _edge_body(ef_ref, ea_ref, w1_ref, w2_ref, w3_ref, w4_ref, c_ref):
    h = _silu(jnp.dot(ef_ref[...], w1_ref[...], preferred_element_type=jnp.float32))
    h = _silu(jnp.dot(h, w2_ref[...], preferred_element_type=jnp.float32))
    h = _silu(jnp.dot(h, w3_ref[...], preferred_element_type=jnp.float32))
    ea = ea_ref[...]
    acc = jnp.dot(h, w4_ref[0], preferred_element_type=jnp.float32) * ea[:, 0:1]
    for a in range(1, A):
        acc = acc + jnp.dot(h, w4_ref[a],
                            preferred_element_type=jnp.float32) * ea[:, a:a + 1]
    c_ref[...] = acc


def _edge_call(ef, ea, w1, w2, w3, w4s):
    grid = (N_EDGES // EB,)
    return pl.pallas_call(
        _edge_body,
        grid=grid,
        in_specs=[
            pl.BlockSpec((EB, 8), lambda i: (i, 0)),
            pl.BlockSpec((EB, A), lambda i: (i, 0)),
            pl.BlockSpec((8, MLP_H), lambda i: (0, 0)),
            pl.BlockSpec((MLP_H, MLP_H), lambda i: (0, 0)),
            pl.BlockSpec((MLP_H, MLP_H), lambda i: (0, 0)),
            pl.BlockSpec((A, MLP_H, D), lambda i: (0, 0, 0)),
        ],
        out_specs=pl.BlockSpec((EB, D), lambda i: (i, 0)),
        out_shape=jax.ShapeDtypeStruct((N_EDGES, D), jnp.float32),
    )(ef, ea, w1, w2, w3, w4s)


@functools.lru_cache(maxsize=1)
def _sc_scatter_call():
    mesh = plsc.VectorSubcoreMesh(
        core_axis_name="c", subcore_axis_name="s",
        num_cores=NC, num_subcores=NS)
    return pl.kernel(
        _sc_scatter_body,
        mesh=mesh,
        out_type=jax.ShapeDtypeStruct((NC, N_NODES, D), jnp.float32),
        scratch_types=[
            pltpu.VMEM((K,), jnp.int32),
            pltpu.VMEM((K,), jnp.int32),
            pltpu.VMEM((K, D), jnp.float32),
            pltpu.VMEM((K, D), jnp.float32),
            pltpu.VMEM_SHARED((N_NODES, D), jnp.float32),
            pltpu.SemaphoreType.DMA,
        ],
    )


def _sc_scatter_body(x_hbm, c_hbm, snd_hbm, rcv_hbm, out_hbm,
                     sidx, ridx, cbuf, xsbuf, acc, sem):
    cid = lax.axis_index("c")
    sid = lax.axis_index("s")
    wid = sid * NC + cid

    def zrow(i, _):
        for j in range(D // 16):
            cbuf[i, pl.ds(j * 16, 16)] = jnp.zeros((16,), jnp.float32)
        return 0
    lax.fori_loop(0, K, zrow, 0)

    nz_mine = (NZCHUNKS - sid + NS - 1) // NS

    def zchunk(t, _):
        base = (sid + t * NS) * ZROWS
        pltpu.sync_copy(cbuf.at[pl.ds(0, ZROWS)], acc.at[pl.ds(base, ZROWS)])
        return 0
    lax.fori_loop(0, nz_mine, zchunk, 0)

    plsc.subcore_barrier()

    n_mine = (N_CHUNKS - wid + NW - 1) // NW

    def chunk_body(t, _):
        base = (wid + t * NW) * K
        pltpu.sync_copy(snd_hbm.at[pl.ds(base, K)], sidx)
        pltpu.sync_copy(rcv_hbm.at[pl.ds(base, K)], ridx)
        pltpu.sync_copy(c_hbm.at[pl.ds(base, K)], cbuf)
        pltpu.async_copy(x_hbm.at[sidx], xsbuf, sem).wait()

        def mrow(i, _):
            for j in range(D // 16):
                sl = pl.ds(j * 16, 16)
                cbuf[i, sl] = cbuf[i, sl] * xsbuf[i, sl]
            return 0
        lax.fori_loop(0, K, mrow, 0)

        pltpu.sync_copy(cbuf, acc.at[ridx], add=True)
        return 0
    lax.fori_loop(0, n_mine, chunk_body, 0)

    plsc.subcore_barrier()

    def drain(t, _):
        base = (sid + t * NS) * ZROWS
        pltpu.sync_copy(acc.at[pl.ds(base, ZROWS)],
                        out_hbm.at[cid, pl.ds(base, ZROWS)])
        return 0
    lax.fori_loop(0, nz_mine, drain, 0)


def _out_body(p_ref, w_ref, o_ref):
    m = p_ref[0] + p_ref[1]
    o_ref[...] = jnp.dot(m, w_ref[...],
                         preferred_element_type=jnp.float32) * (1.0 / AVG_NUM_NEIGHBORS)


def _out_call(partials, w_out):
    grid = (N_NODES // NB,)
    return pl.pallas_call(
        _out_body,
        grid=grid,
        in_specs=[
            pl.BlockSpec((NC, NB, D), lambda i: (0, i, 0)),
            pl.BlockSpec((D, D), lambda i: (0, 0)),
        ],
        out_specs=pl.BlockSpec((NB, D), lambda i: (i, 0)),
        out_shape=jax.ShapeDtypeStruct((N_NODES, D), jnp.float32),
    )(partials, w_out)


def kernel(node_attrs, node_feats, edge_attrs, edge_feats, senders, receivers,
           W_sc, W_lin, W1, W2, W3, W4, W_out):
    snd = senders.astype(jnp.int32)
    rcv = receivers.astype(jnp.int32)
    wsc = jnp.transpose(W_sc.reshape(D, A, HIDDEN), (1, 0, 2))
    w4s = jnp.transpose(W4.reshape(MLP_H, D, A), (2, 0, 1))

    sc, x = _node_call(node_feats, node_attrs, wsc, W_lin)
    c = _edge_call(edge_feats, edge_attrs, W1, W2, W3, w4s)
    partials = _sc_scatter_call()(x, c, snd, rcv)
    message = _out_call(partials, W_out)
    return (message, sc)


# speedup vs baseline: 7.3525x; 1.8607x over previous
import functools

import jax
import jax.numpy as jnp
from jax import lax
from jax.experimental import pallas as pl
from jax.experimental.pallas import tpu as pltpu
from jax.experimental.pallas import tpu_sc as plsc

N_NODES = 10000
N_EDGES = 160000
D = 128
A = 4
HIDDEN = 128
MLP_H = 64
AVG_NUM_NEIGHBORS = 16.0

NC = 2
NS = 16
NW = NC * NS
K = 64
N_CHUNKS = N_EDGES // K
NT_MAX = (N_CHUNKS + NW - 1) // NW
ZROWS = 40
NZCHUNKS = N_NODES // ZROWS

NB = 2000
EB = 3200


def _silu(v):
    return v * jax.nn.sigmoid(v)


def _node_body(nf_ref, na_ref, wsc_ref, wlin_ref, sc_ref, x_ref):
    nf = nf_ref[...]
    na = na_ref[...]
    x_ref[...] = jnp.dot(nf, wlin_ref[...], preferred_element_type=jnp.float32)
    acc = jnp.dot(nf * na[:, 0:1], wsc_ref[0], preferred_element_type=jnp.float32)
    for a in range(1, A):
        acc = acc + jnp.dot(nf * na[:, a:a + 1], wsc_ref[a],
                            preferred_element_type=jnp.float32)
    sc_ref[...] = acc


def _node_call(nf, na, wsc, wlin):
    grid = (N_NODES // NB,)
    return pl.pallas_call(
        _node_body,
        grid=grid,
        in_specs=[
            pl.BlockSpec((NB, D), lambda i: (i, 0)),
            pl.BlockSpec((NB, A), lambda i: (i, 0)),
            pl.BlockSpec((A, D, HIDDEN), lambda i: (0, 0, 0)),
            pl.BlockSpec((D, D), lambda i: (0, 0)),
        ],
        out_specs=[
            pl.BlockSpec((NB, HIDDEN), lambda i: (i, 0)),
            pl.BlockSpec((NB, D), lambda i: (i, 0)),
        ],
        out_shape=[
            jax.ShapeDtypeStruct((N_NODES, HIDDEN), jnp.float32),
            jax.ShapeDtypeStruct((N_NODES, D), jnp.float32),
        ],
    )(nf, na, wsc, wlin)


_TN = (((0,), (0,)), ((), ()))


def _edge_body(eft_ref, eat_ref, w1_ref, w2_ref, w3_ref, w4_ref, c_ref):
    ht = _silu(lax.dot_general(w1_ref[...], eft_ref[...], _TN,
                               preferred_element_type=jnp.float32))
    ht = _silu(lax.dot_general(w2_ref[...], ht, _TN,
                               preferred_element_type=jnp.float32))
    ht = _silu(lax.dot_general(w3_ref[...], ht, _TN,
                               preferred_element_type=jnp.float32))
    eat = eat_ref[...]
    het = jnp.concatenate([ht * eat[a:a + 1, :] for a in range(A)],
                          axis=0)
    cres = lax.dot_general(het.astype(jnp.bfloat16), w4_ref[...], _TN,
                           preferred_element_type=jnp.float32)
    ai = lax.bitcast_convert_type(cres[:, :D // 2].astype(jnp.bfloat16),
                                  jnp.uint16).astype(jnp.int32)
    bi = lax.bitcast_convert_type(cres[:, D // 2:].astype(jnp.bfloat16),
                                  jnp.uint16).astype(jnp.int32)
    c_ref[...] = ai | (bi << 16)


def _edge_call(eft, eat, w1, w2, w3, w4r):
    grid = (N_EDGES // EB,)
    return pl.pallas_call(
        _edge_body,
        grid=grid,
        in_specs=[
            pl.BlockSpec((8, EB), lambda i: (0, i)),
            pl.BlockSpec((A, EB), lambda i: (0, i)),
            pl.BlockSpec((8, MLP_H), lambda i: (0, 0)),
            pl.BlockSpec((MLP_H, MLP_H), lambda i: (0, 0)),
            pl.BlockSpec((MLP_H, MLP_H), lambda i: (0, 0)),
            pl.BlockSpec((A * MLP_H, D), lambda i: (0, 0)),
        ],
        out_specs=pl.BlockSpec((EB, D // 2), lambda i: (i, 0)),
        out_shape=jax.ShapeDtypeStruct((N_EDGES, D // 2), jnp.int32),
    )(eft, eat, w1, w2, w3, w4r)


@functools.lru_cache(maxsize=1)
def _sc_scatter_call():
    mesh = plsc.VectorSubcoreMesh(
        core_axis_name="c", subcore_axis_name="s",
        num_cores=NC, num_subcores=NS)
    return pl.kernel(
        _sc_scatter_body,
        mesh=mesh,
        out_type=jax.ShapeDtypeStruct((NC, N_NODES, D), jnp.float32),
        scratch_types=[
            [pltpu.VMEM((K,), jnp.int32)] * 2,
            [pltpu.VMEM((K,), jnp.int32)] * 2,
            [pltpu.VMEM((K, D // 2), jnp.int32)] * 2,
            [pltpu.VMEM((K, D), jnp.float32)] * 2,
            pltpu.VMEM((K, D), jnp.float32),
            pltpu.VMEM_SHARED((N_NODES, D), jnp.float32),
            [pltpu.SemaphoreType.DMA] * 2,
            [pltpu.SemaphoreType.DMA] * 2,
            [pltpu.SemaphoreType.DMA] * 2,
        ],
        compiler_params=pltpu.CompilerParams(needs_layout_passes=False),
    )


def _sc_scatter_body(x_hbm, c_hbm, snd_hbm, rcv_hbm, out_hbm,
                     sidx, ridx, cbuf, xsbuf, mji, acc, semi, seml, semg):
    cid = lax.axis_index("c")
    sid = lax.axis_index("s")
    wid = sid * NC + cid

    def zrow(i, _):
        for j in range(D // 16):
            mji[i, pl.ds(j * 16, 16)] = jnp.zeros((16,), jnp.float32)
        return 0
    lax.fori_loop(0, K, zrow, 0)

    nz_mine = (NZCHUNKS - sid + NS - 1) // NS

    def zchunk(t, _):
        base = (sid + t * NS) * ZROWS
        pltpu.sync_copy(mji.at[pl.ds(0, ZROWS)], acc.at[pl.ds(base, ZROWS)])
        return 0
    lax.fori_loop(0, nz_mine, zchunk, 0)

    plsc.subcore_barrier()

    n_mine = (N_CHUNKS - wid + NW - 1) // NW

    def issue(t, p):
        base = (wid + t * NW) * K
        pltpu.async_copy(snd_hbm.at[pl.ds(base, K)], sidx[p], semi[p])
        pltpu.async_copy(rcv_hbm.at[pl.ds(base, K)], ridx[p], seml[p])
        pltpu.async_copy(c_hbm.at[pl.ds(base, K)], cbuf[p], seml[p])
        pltpu.make_async_copy(snd_hbm.at[pl.ds(base, K)], sidx[p],
                              semi[p]).wait()
        pltpu.async_copy(x_hbm.at[sidx[p]], xsbuf[p], semg[p])

    issue(0, 0)

    def pair_body(u, _):
        for b in range(2):
            t = 2 * u + b
            p = b

            @pl.when(t + 1 < n_mine)
            def _():
                issue(t + 1, 1 - p)

            @pl.when(t < n_mine)
            def _():
                base = (wid + t * NW) * K
                pltpu.make_async_copy(rcv_hbm.at[pl.ds(base, K)], ridx[p],
                                      seml[p]).wait()
                pltpu.make_async_copy(c_hbm.at[pl.ds(base, K)], cbuf[p],
                                      seml[p]).wait()
                pltpu.make_async_copy(x_hbm.at[pl.ds(0, K)], xsbuf[p],
                                      semg[p]).wait()

                def mrow(i, _):
                    for j in range(D // 32):
                        w = cbuf[p][i, pl.ds(j * 16, 16)]
                        clo = plsc.bitcast(w << 16, jnp.float32)
                        chi = plsc.bitcast(w & jnp.int32(-65536), jnp.float32)
                        mji[i, pl.ds(j * 32, 16)] = (
                            clo * xsbuf[p][i, pl.ds(j * 32, 16)])
                        mji[i, pl.ds(j * 32 + 16, 16)] = (
                            chi * xsbuf[p][i, pl.ds(j * 32 + 16, 16)])
                    return 0
                lax.fori_loop(0, K, mrow, 0)

                pltpu.sync_copy(mji, acc.at[ridx[p]], add=True)
        return 0
    lax.fori_loop(0, (NT_MAX + 1) // 2, pair_body, 0)

    plsc.subcore_barrier()

    def drain(t, _):
        base = (sid + t * NS) * ZROWS
        pltpu.sync_copy(acc.at[pl.ds(base, ZROWS)],
                        out_hbm.at[cid, pl.ds(base, ZROWS)])
        return 0
    lax.fori_loop(0, nz_mine, drain, 0)


def _out_body(p_ref, w_ref, o_ref):
    m = p_ref[0] + p_ref[1]
    o_ref[...] = jnp.dot(m, w_ref[...],
                         preferred_element_type=jnp.float32) * (1.0 / AVG_NUM_NEIGHBORS)


def _out_call(partials, w_out):
    grid = (N_NODES // NB,)
    return pl.pallas_call(
        _out_body,
        grid=grid,
        in_specs=[
            pl.BlockSpec((NC, NB, D), lambda i: (0, i, 0)),
            pl.BlockSpec((D, D), lambda i: (0, 0)),
        ],
        out_specs=pl.BlockSpec((NB, D), lambda i: (i, 0)),
        out_shape=jax.ShapeDtypeStruct((N_NODES, D), jnp.float32),
    )(partials, w_out)


def kernel(node_attrs, node_feats, edge_attrs, edge_feats, senders, receivers,
           W_sc, W_lin, W1, W2, W3, W4, W_out):
    snd = senders.astype(jnp.int32)
    rcv = receivers.astype(jnp.int32)
    wsc = jnp.transpose(W_sc.reshape(D, A, HIDDEN), (1, 0, 2))
    w4r = jnp.transpose(W4.reshape(MLP_H, D, A), (2, 0, 1)).reshape(A * MLP_H, D)
    rho = jnp.concatenate([jnp.arange(0, D, 2), jnp.arange(1, D, 2)])
    k16 = jnp.arange(16)
    omega = (jnp.arange(0, D, 32)[:, None]
             + jnp.concatenate([2 * k16, 2 * k16 + 1])[None, :]).reshape(D)
    w4rp = w4r[:, rho].astype(jnp.bfloat16)
    wlinp = W_lin[:, omega]
    woutp = W_out[omega, :]

    sc, x = _node_call(node_feats, node_attrs, wsc, wlinp)
    c = _edge_call(edge_feats.T, edge_attrs.T, W1, W2, W3, w4rp)
    partials = _sc_scatter_call()(x, c, snd, rcv)
    message = _out_call(partials, woutp)
    return (message, sc)


# speedup vs baseline: 7.7547x; 1.0547x over previous
import functools

import jax
import jax.numpy as jnp
from jax import lax
from jax.experimental import pallas as pl
from jax.experimental.pallas import tpu as pltpu
from jax.experimental.pallas import tpu_sc as plsc

N_NODES = 10000
N_EDGES = 160000
D = 128
A = 4
HIDDEN = 128
MLP_H = 64
AVG_NUM_NEIGHBORS = 16.0

NC = 2
NS = 16
NW = NC * NS
K = 64
N_CHUNKS = N_EDGES // K
NT_MAX = (N_CHUNKS + NW - 1) // NW
ZROWS = 40
NZCHUNKS = N_NODES // ZROWS

NB = 2000
EB = 6400


def _silu(v):
    return v * jax.nn.sigmoid(v)


def _node_body(nf_ref, na_ref, wsc_ref, wlin_ref, sc_ref, x_ref):
    nf = nf_ref[...]
    na = na_ref[...]
    x_ref[...] = jnp.dot(nf, wlin_ref[...], preferred_element_type=jnp.float32)
    acc = jnp.dot(nf * na[:, 0:1], wsc_ref[0], preferred_element_type=jnp.float32)
    for a in range(1, A):
        acc = acc + jnp.dot(nf * na[:, a:a + 1], wsc_ref[a],
                            preferred_element_type=jnp.float32)
    sc_ref[...] = acc


def _node_call(nf, na, wsc, wlin):
    grid = (N_NODES // NB,)
    return pl.pallas_call(
        _node_body,
        grid=grid,
        in_specs=[
            pl.BlockSpec((NB, D), lambda i: (i, 0)),
            pl.BlockSpec((NB, A), lambda i: (i, 0)),
            pl.BlockSpec((A, D, HIDDEN), lambda i: (0, 0, 0)),
            pl.BlockSpec((D, D), lambda i: (0, 0)),
        ],
        out_specs=[
            pl.BlockSpec((NB, HIDDEN), lambda i: (i, 0)),
            pl.BlockSpec((NB, D), lambda i: (i, 0)),
        ],
        out_shape=[
            jax.ShapeDtypeStruct((N_NODES, HIDDEN), jnp.float32),
            jax.ShapeDtypeStruct((N_NODES, D), jnp.float32),
        ],
    )(nf, na, wsc, wlin)


_TN = (((0,), (0,)), ((), ()))


def _edge_body(eft_ref, eat_ref, w1_ref, w2_ref, w3_ref, w4_ref, c_ref):
    ht = _silu(lax.dot_general(w1_ref[...], eft_ref[...], _TN,
                               preferred_element_type=jnp.float32))
    ht = _silu(lax.dot_general(w2_ref[...], ht, _TN,
                               preferred_element_type=jnp.float32))
    ht = _silu(lax.dot_general(w3_ref[...], ht, _TN,
                               preferred_element_type=jnp.float32))
    eat = eat_ref[...]
    het = jnp.concatenate([ht * eat[a:a + 1, :] for a in range(A)],
                          axis=0)
    cres = lax.dot_general(het.astype(jnp.bfloat16), w4_ref[...], _TN,
                           preferred_element_type=jnp.float32)
    ai = lax.bitcast_convert_type(cres[:, :D // 2].astype(jnp.bfloat16),
                                  jnp.uint16).astype(jnp.int32)
    bi = lax.bitcast_convert_type(cres[:, D // 2:].astype(jnp.bfloat16),
                                  jnp.uint16).astype(jnp.int32)
    c_ref[...] = ai | (bi << 16)


def _edge_call(eft, eat, w1, w2, w3, w4r):
    grid = (N_EDGES // EB,)
    return pl.pallas_call(
        _edge_body,
        grid=grid,
        in_specs=[
            pl.BlockSpec((8, EB), lambda i: (0, i)),
            pl.BlockSpec((A, EB), lambda i: (0, i)),
            pl.BlockSpec((8, MLP_H), lambda i: (0, 0)),
            pl.BlockSpec((MLP_H, MLP_H), lambda i: (0, 0)),
            pl.BlockSpec((MLP_H, MLP_H), lambda i: (0, 0)),
            pl.BlockSpec((A * MLP_H, D), lambda i: (0, 0)),
        ],
        out_specs=pl.BlockSpec((EB, D // 2), lambda i: (i, 0)),
        out_shape=jax.ShapeDtypeStruct((N_EDGES, D // 2), jnp.int32),
    )(eft, eat, w1, w2, w3, w4r)


@functools.lru_cache(maxsize=1)
def _sc_scatter_call():
    mesh = plsc.VectorSubcoreMesh(
        core_axis_name="c", subcore_axis_name="s",
        num_cores=NC, num_subcores=NS)
    return pl.kernel(
        _sc_scatter_body,
        mesh=mesh,
        out_type=jax.ShapeDtypeStruct((NC, N_NODES, D), jnp.float32),
        scratch_types=[
            [pltpu.VMEM((K,), jnp.int32)] * 2,
            [pltpu.VMEM((K,), jnp.int32)] * 2,
            [pltpu.VMEM((K, D // 2), jnp.int32)] * 2,
            [pltpu.VMEM((K, D), jnp.float32)] * 2,
            pltpu.VMEM((K, D), jnp.float32),
            pltpu.VMEM_SHARED((N_NODES, D), jnp.float32),
            [pltpu.SemaphoreType.DMA] * 2,
            [pltpu.SemaphoreType.DMA] * 2,
            [pltpu.SemaphoreType.DMA] * 2,
        ],
        compiler_params=pltpu.CompilerParams(needs_layout_passes=False),
    )


def _sc_scatter_body(x_hbm, c_hbm, snd_hbm, rcv_hbm, out_hbm,
                     sidx, ridx, cbuf, xsbuf, mji, acc, semi, seml, semg):
    cid = lax.axis_index("c")
    sid = lax.axis_index("s")
    wid = sid * NC + cid

    def zrow(i, _):
        for j in range(D // 16):
            mji[i, pl.ds(j * 16, 16)] = jnp.zeros((16,), jnp.float32)
        return 0
    lax.fori_loop(0, K, zrow, 0)

    nz_mine = (NZCHUNKS - sid + NS - 1) // NS

    def zchunk(t, _):
        base = (sid + t * NS) * ZROWS
        pltpu.sync_copy(mji.at[pl.ds(0, ZROWS)], acc.at[pl.ds(base, ZROWS)])
        return 0
    lax.fori_loop(0, nz_mine, zchunk, 0)

    plsc.subcore_barrier()

    n_mine = (N_CHUNKS - wid + NW - 1) // NW

    def issue(t, p):
        base = (wid + t * NW) * K
        pltpu.async_copy(snd_hbm.at[pl.ds(base, K)], sidx[p], semi[p])
        pltpu.async_copy(rcv_hbm.at[pl.ds(base, K)], ridx[p], seml[p])
        pltpu.async_copy(c_hbm.at[pl.ds(base, K)], cbuf[p], seml[p])
        pltpu.make_async_copy(snd_hbm.at[pl.ds(base, K)], sidx[p],
                              semi[p]).wait()
        pltpu.async_copy(x_hbm.at[sidx[p]], xsbuf[p], semg[p])

    issue(0, 0)

    def pair_body(u, _):
        for b in range(2):
            t = 2 * u + b
            p = b

            @pl.when(t + 1 < n_mine)
            def _():
                issue(t + 1, 1 - p)

            @pl.when(t < n_mine)
            def _():
                base = (wid + t * NW) * K
                pltpu.make_async_copy(rcv_hbm.at[pl.ds(base, K)], ridx[p],
                                      seml[p]).wait()
                pltpu.make_async_copy(c_hbm.at[pl.ds(base, K)], cbuf[p],
                                      seml[p]).wait()
                pltpu.make_async_copy(x_hbm.at[pl.ds(0, K)], xsbuf[p],
                                      semg[p]).wait()

                def mrow(i, _):
                    for j in range(D // 32):
                        w = cbuf[p][i, pl.ds(j * 16, 16)]
                        clo = plsc.bitcast(w << 16, jnp.float32)
                        chi = plsc.bitcast(w & jnp.int32(-65536), jnp.float32)
                        mji[i, pl.ds(j * 32, 16)] = (
                            clo * xsbuf[p][i, pl.ds(j * 32, 16)])
                        mji[i, pl.ds(j * 32 + 16, 16)] = (
                            chi * xsbuf[p][i, pl.ds(j * 32 + 16, 16)])
                    return 0
                lax.fori_loop(0, K, mrow, 0)

                pltpu.sync_copy(mji, acc.at[ridx[p]], add=True)
        return 0
    lax.fori_loop(0, (NT_MAX + 1) // 2, pair_body, 0)

    plsc.subcore_barrier()

    def drain(t, _):
        base = (sid + t * NS) * ZROWS
        pltpu.sync_copy(acc.at[pl.ds(base, ZROWS)],
                        out_hbm.at[cid, pl.ds(base, ZROWS)])
        return 0
    lax.fori_loop(0, nz_mine, drain, 0)


def _out_body(p_ref, w_ref, o_ref):
    m = p_ref[0] + p_ref[1]
    o_ref[...] = jnp.dot(m, w_ref[...],
                         preferred_element_type=jnp.float32) * (1.0 / AVG_NUM_NEIGHBORS)


def _out_call(partials, w_out):
    grid = (N_NODES // NB,)
    return pl.pallas_call(
        _out_body,
        grid=grid,
        in_specs=[
            pl.BlockSpec((NC, NB, D), lambda i: (0, i, 0)),
            pl.BlockSpec((D, D), lambda i: (0, 0)),
        ],
        out_specs=pl.BlockSpec((NB, D), lambda i: (i, 0)),
        out_shape=jax.ShapeDtypeStruct((N_NODES, D), jnp.float32),
    )(partials, w_out)


def kernel(node_attrs, node_feats, edge_attrs, edge_feats, senders, receivers,
           W_sc, W_lin, W1, W2, W3, W4, W_out):
    snd = senders.astype(jnp.int32)
    rcv = receivers.astype(jnp.int32)
    wsc = jnp.transpose(W_sc.reshape(D, A, HIDDEN), (1, 0, 2))
    w4r = jnp.transpose(W4.reshape(MLP_H, D, A), (2, 0, 1)).reshape(A * MLP_H, D)
    rho = jnp.concatenate([jnp.arange(0, D, 2), jnp.arange(1, D, 2)])
    k16 = jnp.arange(16)
    omega = (jnp.arange(0, D, 32)[:, None]
             + jnp.concatenate([2 * k16, 2 * k16 + 1])[None, :]).reshape(D)
    w4rp = w4r[:, rho].astype(jnp.bfloat16)
    wlinp = W_lin[:, omega]
    woutp = W_out[omega, :]

    sc, x = _node_call(node_feats, node_attrs, wsc, wlinp)
    c = _edge_call(edge_feats.T, edge_attrs.T, W1, W2, W3, w4rp)
    partials = _sc_scatter_call()(x, c, snd, rcv)
    message = _out_call(partials, woutp)
    return (message, sc)
